# baseline (speedup 1.0000x reference)
import dataclasses
import functools

import jax
import jax.numpy as jnp
from jax import lax
from jax.experimental import pallas as pl
from jax.experimental.pallas import tpu as pltpu
from jax.experimental.pallas import tpu_sc as plsc

N_NODES = 10000
N_EDGES = 320000
NUM_ATOMS = 128

NC = 2
NS = 16
NW = NC * NS
LANES = 16
EPW = N_EDGES // NW
G = 80
NCHUNK = EPW // G
GROUPS = G // LANES


def _prescale_body(codes_ref, w_ref, scaled_ref, raw_ref):
    c = codes_ref[...]
    scaled_ref[...] = (c * w_ref[...]).astype(jnp.bfloat16)
    raw_ref[...] = c.astype(jnp.bfloat16)


def _prescale(codes, w):
    return pl.pallas_call(
        _prescale_body,
        out_shape=[
            jax.ShapeDtypeStruct((N_NODES, NUM_ATOMS), jnp.bfloat16),
            jax.ShapeDtypeStruct((N_NODES, NUM_ATOMS), jnp.bfloat16),
        ],
    )(codes, w.reshape(1, NUM_ATOMS))


def _edge_score_sc(table_scaled, table_raw, src_idx, dst_idx):
    mesh = plsc.VectorSubcoreMesh(core_axis_name="c", subcore_axis_name="s")
    cp = pltpu.CompilerParams()
    if "needs_layout_passes" in pltpu.CompilerParams.__dataclass_fields__:
        cp = dataclasses.replace(cp, needs_layout_passes=False)
    if "use_tc_tiling_on_sc" in pltpu.CompilerParams.__dataclass_fields__:
        cp = dataclasses.replace(cp, use_tc_tiling_on_sc=False)

    @functools.partial(
        pl.kernel,
        mesh=mesh,
        compiler_params=cp,
        out_type=jax.ShapeDtypeStruct((N_EDGES,), jnp.float32),
        scratch_types=[
            pltpu.VMEM((G,), jnp.int32),
            pltpu.VMEM((G,), jnp.int32),
            pltpu.VMEM((G,), jnp.int32),
            pltpu.VMEM((G,), jnp.int32),
            pltpu.VMEM((G, NUM_ATOMS // 2), jnp.int32),
            pltpu.VMEM((G, NUM_ATOMS // 2), jnp.int32),
            pltpu.VMEM((G, NUM_ATOMS // 2), jnp.int32),
            pltpu.VMEM((G, NUM_ATOMS // 2), jnp.int32),
            pltpu.VMEM((G * LANES,), jnp.float32),
            pltpu.VMEM((G,), jnp.float32),
            pltpu.VMEM((G,), jnp.float32),
            pltpu.SemaphoreType.DMA,
            pltpu.SemaphoreType.DMA,
            pltpu.SemaphoreType.DMA,
            pltpu.SemaphoreType.DMA,
        ],
    )
    def k(ts_hbm, tr_hbm, sidx_hbm, didx_hbm, out_hbm,
          sidxA, didxA, sidxB, didxB, srA, drA, srB, drB, part_v,
          outA, outB, semgA, semgB, semoA, semoB):
        wid = lax.axis_index("s") * NC + lax.axis_index("c")
        tile_base = wid * EPW
        bufs = {
            0: (sidxA, didxA, srA, drA, outA, semgA, semoA),
            1: (sidxB, didxB, srB, drB, outB, semgB, semoB),
        }

        def issue(chunk, b):
            sidx, didx, sr, dr, _, semg, _ = bufs[b]
            base = tile_base + chunk * G
            pltpu.sync_copy(sidx_hbm.at[pl.ds(base, G)], sidx)
            pltpu.sync_copy(didx_hbm.at[pl.ds(base, G)], didx)
            pltpu.async_copy(ts_hbm.at[sidx], sr, semg)
            pltpu.async_copy(tr_hbm.at[didx], dr, semg)

        def compute(chunk, b):
            sidx, didx, sr, dr, outb, semg, semo = bufs[b]
            pltpu.make_async_copy(ts_hbm.at[sidx], sr, semg).wait()
            pltpu.make_async_copy(tr_hbm.at[didx], dr, semg).wait()

            @pl.when(chunk >= 2)
            def _():
                pltpu.make_async_copy(
                    outb, out_hbm.at[pl.ds(tile_base, G)], semo).wait()

            @pl.loop(0, G)
            def _(e):
                def ld(ref, j):
                    return plsc.bitcast(
                        ref[e, pl.ds(j * LANES, LANES)], jnp.bfloat16)

                acc = ld(sr, 0) * ld(dr, 0)
                for j in range(1, NUM_ATOMS // (2 * LANES)):
                    acc = jnp.maximum(acc, ld(sr, j) * ld(dr, j))
                lo, hi = plsc.unpack(acc, format=plsc.PackFormat.INTERLEAVED)
                part_v[pl.ds(e * LANES, LANES)] = jnp.maximum(lo, hi)

            iota = lax.iota(jnp.int32, LANES)

            @pl.loop(0, GROUPS)
            def _(t):
                col = t * (LANES * LANES) + iota * LANES
                m = plsc.load_gather(part_v, [col])
                for l in range(1, LANES):
                    m = jnp.maximum(m, plsc.load_gather(part_v, [col + l]))
                outb[pl.ds(t * LANES, LANES)] = 1.0 / (1.0 + jnp.exp(-m))

            pltpu.async_copy(
                outb, out_hbm.at[pl.ds(tile_base + chunk * G, G)], semo)

        issue(0, 0)

        @pl.loop(0, NCHUNK, step=2)
        def _(c):
            @pl.when(c + 1 < NCHUNK)
            def _():
                issue(c + 1, 1)

            compute(c, 0)

            @pl.when(c + 2 < NCHUNK)
            def _():
                issue(c + 2, 0)

            @pl.when(c + 1 < NCHUNK)
            def _():
                compute(c + 1, 1)

        pltpu.make_async_copy(outA, out_hbm.at[pl.ds(tile_base, G)], semoA).wait()
        pltpu.make_async_copy(outB, out_hbm.at[pl.ds(tile_base, G)], semoB).wait()

    return k(table_scaled, table_raw, src_idx, dst_idx)


def kernel(sparse_codes, edge_index, pattern_weights):
    scaled, raw = _prescale(sparse_codes, pattern_weights)
    scaled = lax.bitcast_convert_type(
        scaled.reshape(N_NODES, NUM_ATOMS // 2, 2), jnp.int32)
    raw = lax.bitcast_convert_type(
        raw.reshape(N_NODES, NUM_ATOMS // 2, 2), jnp.int32)
    src_idx = edge_index[0].astype(jnp.int32)
    dst_idx = edge_index[1].astype(jnp.int32)
    return _edge_score_sc(scaled, raw, src_idx, dst_idx)


# speedup vs baseline: 1.4293x; 1.4293x over previous
import dataclasses
import functools

import jax
import jax.numpy as jnp
from jax import lax
from jax.experimental import pallas as pl
from jax.experimental.pallas import tpu as pltpu
from jax.experimental.pallas import tpu_sc as plsc

N_NODES = 10000
N_EDGES = 320000
NUM_ATOMS = 128

NC = 2
NS = 16
NW = NC * NS
LANES = 16
EPW = N_EDGES // NW
G = 80
NCHUNK = EPW // G
GROUPS = G // LANES


def _prescale_body(codes_ref, w_ref, scaled_ref, raw_ref):
    c = codes_ref[...]
    scaled_ref[...] = (c * w_ref[...]).astype(jnp.bfloat16)
    raw_ref[...] = c.astype(jnp.bfloat16)


def _prescale(codes, w):
    return pl.pallas_call(
        _prescale_body,
        out_shape=[
            jax.ShapeDtypeStruct((N_NODES, NUM_ATOMS), jnp.bfloat16),
            jax.ShapeDtypeStruct((N_NODES, NUM_ATOMS), jnp.bfloat16),
        ],
    )(codes, w.reshape(1, NUM_ATOMS))


def _edge_score_sc(table_scaled, table_raw, src_idx, dst_idx):
    mesh = plsc.VectorSubcoreMesh(core_axis_name="c", subcore_axis_name="s")
    cp = pltpu.CompilerParams()
    if "needs_layout_passes" in pltpu.CompilerParams.__dataclass_fields__:
        cp = dataclasses.replace(cp, needs_layout_passes=False)
    if "use_tc_tiling_on_sc" in pltpu.CompilerParams.__dataclass_fields__:
        cp = dataclasses.replace(cp, use_tc_tiling_on_sc=False)

    @functools.partial(
        pl.kernel,
        mesh=mesh,
        compiler_params=cp,
        out_type=jax.ShapeDtypeStruct((N_EDGES,), jnp.float32),
        scratch_types=[
            pltpu.VMEM((NCHUNK, G), jnp.int32),
            pltpu.VMEM((NCHUNK, G), jnp.int32),
            pltpu.VMEM((G, NUM_ATOMS // 2), jnp.int32),
            pltpu.VMEM((G, NUM_ATOMS // 2), jnp.int32),
            pltpu.VMEM((G, NUM_ATOMS // 2), jnp.int32),
            pltpu.VMEM((G, NUM_ATOMS // 2), jnp.int32),
            pltpu.VMEM((G * LANES,), jnp.float32),
            pltpu.VMEM((EPW,), jnp.float32),
            pltpu.SemaphoreType.DMA,
            pltpu.SemaphoreType.DMA,
            pltpu.SemaphoreType.DMA,
        ],
    )
    def k(ts_hbm, tr_hbm, sidx_hbm, didx_hbm, out_hbm,
          sidx_v, didx_v, srA, drA, srB, drB, part_v,
          out_v, semgA, semgB, semo):
        wid = lax.axis_index("s") * NC + lax.axis_index("c")
        tile_base = wid * EPW
        bufs = {0: (srA, drA, semgA), 1: (srB, drB, semgB)}

        pltpu.sync_copy(sidx_hbm.at[wid], sidx_v)
        pltpu.sync_copy(didx_hbm.at[wid], didx_v)

        def issue(chunk, b):
            sr, dr, semg = bufs[b]
            pltpu.async_copy(ts_hbm.at[sidx_v.at[chunk]], sr, semg)
            pltpu.async_copy(tr_hbm.at[didx_v.at[chunk]], dr, semg)

        def compute(chunk, b):
            sr, dr, semg = bufs[b]
            pltpu.make_async_copy(ts_hbm.at[sidx_v.at[chunk]], sr, semg).wait()
            pltpu.make_async_copy(tr_hbm.at[didx_v.at[chunk]], dr, semg).wait()

            @pl.loop(0, G)
            def _(e):
                def ld(ref, j):
                    return plsc.bitcast(
                        ref[e, pl.ds(j * LANES, LANES)], jnp.bfloat16)

                acc = ld(sr, 0) * ld(dr, 0)
                for j in range(1, NUM_ATOMS // (2 * LANES)):
                    acc = jnp.maximum(acc, ld(sr, j) * ld(dr, j))
                lo, hi = plsc.unpack(acc, format=plsc.PackFormat.INTERLEAVED)
                part_v[pl.ds(e * LANES, LANES)] = jnp.maximum(lo, hi)

            iota = lax.iota(jnp.int32, LANES)

            @pl.loop(0, GROUPS)
            def _(t):
                col = t * (LANES * LANES) + iota * LANES
                m = plsc.load_gather(part_v, [col])
                for l in range(1, LANES):
                    m = jnp.maximum(m, plsc.load_gather(part_v, [col + l]))
                out_v[pl.ds(chunk * G + t * LANES, LANES)] = (
                    1.0 / (1.0 + jnp.exp(-m)))

        issue(0, 0)

        @pl.loop(0, NCHUNK, step=2)
        def _(c):
            @pl.when(c + 1 < NCHUNK)
            def _():
                issue(c + 1, 1)

            compute(c, 0)

            @pl.when(c + 2 < NCHUNK)
            def _():
                issue(c + 2, 0)

            @pl.when(c + 1 < NCHUNK)
            def _():
                compute(c + 1, 1)

        pltpu.async_copy(out_v, out_hbm.at[pl.ds(tile_base, EPW)], semo).wait()

    return k(table_scaled, table_raw, src_idx, dst_idx)


def kernel(sparse_codes, edge_index, pattern_weights):
    scaled, raw = _prescale(sparse_codes, pattern_weights)
    scaled = lax.bitcast_convert_type(
        scaled.reshape(N_NODES, NUM_ATOMS // 2, 2), jnp.int32)
    raw = lax.bitcast_convert_type(
        raw.reshape(N_NODES, NUM_ATOMS // 2, 2), jnp.int32)
    src_idx = edge_index[0].astype(jnp.int32).reshape(NW, NCHUNK, G)
    dst_idx = edge_index[1].astype(jnp.int32).reshape(NW, NCHUNK, G)
    return _edge_score_sc(scaled, raw, src_idx, dst_idx)


# speedup vs baseline: 1.9492x; 1.3637x over previous
import dataclasses
import functools

import jax
import jax.numpy as jnp
from jax import lax
from jax.experimental import pallas as pl
from jax.experimental.pallas import tpu as pltpu
from jax.experimental.pallas import tpu_sc as plsc

N_NODES = 10000
N_EDGES = 320000
NUM_ATOMS = 128

NC = 2
NS = 16
NW = NC * NS
LANES = 16
EPW = N_EDGES // NW
G = 80
NCHUNK = EPW // G
GROUPS = G // LANES


def _pack_pair(x):
    lo = jax.lax.bitcast_convert_type(
        x[:, : NUM_ATOMS // 2].astype(jnp.bfloat16), jnp.uint16
    ).astype(jnp.uint32)
    hi = jax.lax.bitcast_convert_type(
        x[:, NUM_ATOMS // 2 :].astype(jnp.bfloat16), jnp.uint16
    ).astype(jnp.uint32)
    return (lo | (hi << 16)).astype(jnp.int32)


def _prescale_body(codes_ref, w_ref, scaled_ref, raw_ref):
    c = codes_ref[...]
    scaled_ref[...] = _pack_pair(c * w_ref[...])
    raw_ref[...] = _pack_pair(c)


def _prescale(codes, w):
    return pl.pallas_call(
        _prescale_body,
        out_shape=[
            jax.ShapeDtypeStruct((N_NODES, NUM_ATOMS // 2), jnp.int32),
            jax.ShapeDtypeStruct((N_NODES, NUM_ATOMS // 2), jnp.int32),
        ],
    )(codes, w.reshape(1, NUM_ATOMS))


def _edge_score_sc(table_scaled, table_raw, idx):
    mesh = plsc.VectorSubcoreMesh(core_axis_name="c", subcore_axis_name="s")
    cp = pltpu.CompilerParams()
    if "needs_layout_passes" in pltpu.CompilerParams.__dataclass_fields__:
        cp = dataclasses.replace(cp, needs_layout_passes=False)
    if "use_tc_tiling_on_sc" in pltpu.CompilerParams.__dataclass_fields__:
        cp = dataclasses.replace(cp, use_tc_tiling_on_sc=False)

    @functools.partial(
        pl.kernel,
        mesh=mesh,
        compiler_params=cp,
        out_type=jax.ShapeDtypeStruct((N_EDGES,), jnp.float32),
        scratch_types=[
            pltpu.VMEM((NCHUNK, G), jnp.int32),
            pltpu.VMEM((NCHUNK, G), jnp.int32),
            pltpu.VMEM((G, NUM_ATOMS // 2), jnp.int32),
            pltpu.VMEM((G, NUM_ATOMS // 2), jnp.int32),
            pltpu.VMEM((G, NUM_ATOMS // 2), jnp.int32),
            pltpu.VMEM((G, NUM_ATOMS // 2), jnp.int32),
            pltpu.VMEM((G * LANES,), jnp.float32),
            pltpu.VMEM((EPW,), jnp.float32),
            pltpu.SemaphoreType.DMA,
            pltpu.SemaphoreType.DMA,
            pltpu.SemaphoreType.DMA,
        ],
    )
    def k(ts_hbm, tr_hbm, idx_hbm, out_hbm,
          sidx_v, didx_v, srA, drA, srB, drB, part_v,
          out_v, semgA, semgB, semo):
        wid = lax.axis_index("s") * NC + lax.axis_index("c")
        tile_base = wid * EPW
        bufs = {0: (srA, drA, semgA), 1: (srB, drB, semgB)}

        pltpu.sync_copy(idx_hbm.at[0, wid], sidx_v)
        pltpu.sync_copy(idx_hbm.at[1, wid], didx_v)

        def issue(chunk, b):
            sr, dr, semg = bufs[b]
            pltpu.async_copy(ts_hbm.at[sidx_v.at[chunk]], sr, semg)
            pltpu.async_copy(tr_hbm.at[didx_v.at[chunk]], dr, semg)

        def compute(chunk, b):
            sr, dr, semg = bufs[b]
            pltpu.make_async_copy(ts_hbm.at[sidx_v.at[chunk]], sr, semg).wait()
            pltpu.make_async_copy(tr_hbm.at[didx_v.at[chunk]], dr, semg).wait()

            @pl.loop(0, G)
            def _(e):
                def ld(ref, j):
                    return plsc.bitcast(
                        ref[e, pl.ds(j * LANES, LANES)], jnp.bfloat16)

                acc = ld(sr, 0) * ld(dr, 0)
                for j in range(1, NUM_ATOMS // (2 * LANES)):
                    acc = jnp.maximum(acc, ld(sr, j) * ld(dr, j))
                lo, hi = plsc.unpack(acc, format=plsc.PackFormat.INTERLEAVED)
                part_v[pl.ds(e * LANES, LANES)] = jnp.maximum(lo, hi)

            iota = lax.iota(jnp.int32, LANES)

            @pl.loop(0, GROUPS)
            def _(t):
                col = t * (LANES * LANES) + iota * LANES
                m = plsc.load_gather(part_v, [col])
                for l in range(1, LANES):
                    m = jnp.maximum(m, plsc.load_gather(part_v, [col + l]))
                out_v[pl.ds(chunk * G + t * LANES, LANES)] = (
                    1.0 / (1.0 + jnp.exp(-m)))

        issue(0, 0)

        @pl.loop(0, NCHUNK, step=2)
        def _(c):
            @pl.when(c + 1 < NCHUNK)
            def _():
                issue(c + 1, 1)

            compute(c, 0)

            @pl.when(c + 2 < NCHUNK)
            def _():
                issue(c + 2, 0)

            @pl.when(c + 1 < NCHUNK)
            def _():
                compute(c + 1, 1)

        pltpu.async_copy(out_v, out_hbm.at[pl.ds(tile_base, EPW)], semo).wait()

    return k(table_scaled, table_raw, idx)


def kernel(sparse_codes, edge_index, pattern_weights):
    scaled, raw = _prescale(sparse_codes, pattern_weights)
    idx = edge_index.astype(jnp.int32).reshape(2, NW, NCHUNK, G)
    return _edge_score_sc(scaled, raw, idx)


# speedup vs baseline: 2.3886x; 1.2255x over previous
import dataclasses
import functools

import jax
import jax.numpy as jnp
from jax import lax
from jax.experimental import pallas as pl
from jax.experimental.pallas import tpu as pltpu
from jax.experimental.pallas import tpu_sc as plsc

N_NODES = 10000
N_EDGES = 320000
NUM_ATOMS = 128

NC = 2
NS = 16
NW = NC * NS
LANES = 16
EPW = N_EDGES // NW
G = 80
NCHUNK = EPW // G
GROUPS = G // LANES


def _pack_pair(x):
    lo = jax.lax.bitcast_convert_type(
        x[:, : NUM_ATOMS // 2].astype(jnp.bfloat16), jnp.uint16
    ).astype(jnp.uint32)
    hi = jax.lax.bitcast_convert_type(
        x[:, NUM_ATOMS // 2 :].astype(jnp.bfloat16), jnp.uint16
    ).astype(jnp.uint32)
    return (lo | (hi << 16)).astype(jnp.int32)


def _prescale_body(codes_ref, w_ref, scaled_ref, raw_ref):
    c = codes_ref[...]
    scaled_ref[...] = _pack_pair(c * w_ref[...])
    raw_ref[...] = _pack_pair(c)


def _prescale(codes, w):
    return pl.pallas_call(
        _prescale_body,
        out_shape=[
            jax.ShapeDtypeStruct((N_NODES, NUM_ATOMS // 2), jnp.int32),
            jax.ShapeDtypeStruct((N_NODES, NUM_ATOMS // 2), jnp.int32),
        ],
    )(codes, w.reshape(1, NUM_ATOMS))


def _edge_score_sc(table_scaled, table_raw, idx):
    mesh = plsc.VectorSubcoreMesh(core_axis_name="c", subcore_axis_name="s")
    cp = pltpu.CompilerParams()
    if "needs_layout_passes" in pltpu.CompilerParams.__dataclass_fields__:
        cp = dataclasses.replace(cp, needs_layout_passes=False)
    if "use_tc_tiling_on_sc" in pltpu.CompilerParams.__dataclass_fields__:
        cp = dataclasses.replace(cp, use_tc_tiling_on_sc=False)

    @functools.partial(
        pl.kernel,
        mesh=mesh,
        compiler_params=cp,
        out_type=jax.ShapeDtypeStruct((N_EDGES,), jnp.float32),
        scratch_types=[
            pltpu.VMEM((NCHUNK, G), jnp.int32),
            pltpu.VMEM((NCHUNK, G), jnp.int32),
            pltpu.VMEM((G, NUM_ATOMS // 2), jnp.int32),
            pltpu.VMEM((G, NUM_ATOMS // 2), jnp.int32),
            pltpu.VMEM((G, NUM_ATOMS // 2), jnp.int32),
            pltpu.VMEM((G, NUM_ATOMS // 2), jnp.int32),
            pltpu.VMEM((G * LANES,), jnp.float32),
            pltpu.VMEM((EPW,), jnp.float32),
            pltpu.SemaphoreType.DMA,
            pltpu.SemaphoreType.DMA,
            pltpu.SemaphoreType.DMA,
        ],
    )
    def k(ts_hbm, tr_hbm, idx_hbm, out_hbm,
          sidx_v, didx_v, srA, drA, srB, drB, part_v,
          out_v, semgA, semgB, semo):
        wid = lax.axis_index("s") * NC + lax.axis_index("c")
        tile_base = wid * EPW
        bufs = {0: (srA, drA, semgA), 1: (srB, drB, semgB)}

        pltpu.sync_copy(idx_hbm.at[0, wid], sidx_v)
        pltpu.sync_copy(idx_hbm.at[1, wid], didx_v)

        def issue(chunk, b):
            sr, dr, semg = bufs[b]
            pltpu.async_copy(ts_hbm.at[sidx_v.at[chunk]], sr, semg)
            pltpu.async_copy(tr_hbm.at[didx_v.at[chunk]], dr, semg)

        def compute(chunk, b):
            sr, dr, semg = bufs[b]
            pltpu.make_async_copy(ts_hbm.at[sidx_v.at[chunk]], sr, semg).wait()
            pltpu.make_async_copy(tr_hbm.at[didx_v.at[chunk]], dr, semg).wait()

            @plsc.parallel_loop(0, G, step=1, unroll=4)
            def _(e):
                def ld(ref, j):
                    return plsc.bitcast(
                        ref[e, pl.ds(j * LANES, LANES)], jnp.bfloat16)

                acc = ld(sr, 0) * ld(dr, 0)
                for j in range(1, NUM_ATOMS // (2 * LANES)):
                    acc = jnp.maximum(acc, ld(sr, j) * ld(dr, j))
                lo, hi = plsc.unpack(acc, format=plsc.PackFormat.INTERLEAVED)
                part_v[pl.ds(e * LANES, LANES)] = jnp.maximum(lo, hi)

            iota = lax.iota(jnp.int32, LANES)

            @plsc.parallel_loop(0, GROUPS, step=1, unroll=2)
            def _(t):
                col = t * (LANES * LANES) + iota * LANES
                m = plsc.load_gather(part_v, [col])
                for l in range(1, LANES):
                    m = jnp.maximum(m, plsc.load_gather(part_v, [col + l]))
                out_v[pl.ds(chunk * G + t * LANES, LANES)] = (
                    1.0 / (1.0 + jnp.exp(-m)))

        issue(0, 0)

        @pl.loop(0, NCHUNK, step=2)
        def _(c):
            @pl.when(c + 1 < NCHUNK)
            def _():
                issue(c + 1, 1)

            compute(c, 0)

            @pl.when(c + 2 < NCHUNK)
            def _():
                issue(c + 2, 0)

            @pl.when(c + 1 < NCHUNK)
            def _():
                compute(c + 1, 1)

        pltpu.async_copy(out_v, out_hbm.at[pl.ds(tile_base, EPW)], semo).wait()

    return k(table_scaled, table_raw, idx)


def kernel(sparse_codes, edge_index, pattern_weights):
    scaled, raw = _prescale(sparse_codes, pattern_weights)
    idx = edge_index.astype(jnp.int32).reshape(2, NW, NCHUNK, G)
    return _edge_score_sc(scaled, raw, idx)
